# baseline (speedup 1.0000x reference)
import functools

import numpy as np

import jax
import jax.numpy as jnp
from jax import lax
from jax.experimental import pallas as pl
from jax.experimental.pallas import tpu as pltpu
from jax.experimental.pallas import tpu_sc as plsc

_D = 32
_SEQ = 200
_BATCH = 4096
_NW = 32
_BLK = _BATCH // _NW
_S = 5
_NIT = _SEQ // _S
_ROWS = _S * _BLK


_V = 1000000
_TCOLS = _V // 128
_TAIL = _V - _TCOLS * 128


def _make_retile():
    mesh = plsc.VectorSubcoreMesh(core_axis_name="c", subcore_axis_name="s")
    ntc = _TCOLS + 1
    nit = 246

    @functools.partial(
        pl.kernel,
        mesh=mesh,
        out_type=jax.ShapeDtypeStruct((_V * _D // 128, 128), jnp.float32),
        compiler_params=pltpu.CompilerParams(needs_layout_passes=False),
        scratch_types=[
            pltpu.VMEM((_D, 129), jnp.float32),
            pltpu.VMEM((_D, 129), jnp.float32),
            pltpu.VMEM((128, 33), jnp.float32),
            pltpu.VMEM((128, 33), jnp.float32),
            pltpu.VMEM((32, 128), jnp.float32),
            pltpu.VMEM((32, 128), jnp.float32),
            pltpu.VMEM((16, 128), jnp.float32),
            pltpu.SemaphoreType.DMA,
            pltpu.SemaphoreType.DMA,
            pltpu.SemaphoreType.DMA,
            pltpu.SemaphoreType.DMA,
        ],
    )
    def retile_kernel(wt_hbm, tail_hbm, lin_hbm, b0, b1, m0, m1, t0, t1,
                      tbuf, sr0, sr1, sw0, sw1):
        wid = lax.axis_index("s") * 2 + lax.axis_index("c")
        buf = (b0, b1)
        tmid = (m0, m1)
        tout = (t0, t1)
        sr = (sr0, sr1)
        sw = (sw0, sw1)
        lane = jax.lax.iota(jnp.int32, 16)
        rlv = [lane + r8 * 16 for r8 in range(8)]
        zeros16 = jnp.zeros((16,), jnp.int32)

        def read_start(k, p):
            tc = wid + 32 * k

            @pl.when(tc < _TCOLS)
            def _():
                for c_abs in range(_D):
                    pltpu.async_copy(
                        wt_hbm.at[c_abs, pl.ds(tc * 128, 128)],
                        buf[p].at[c_abs, pl.ds(0, 128)], sr[p])

            @pl.when(tc == _TCOLS)
            def _():
                for i in range(2):
                    pltpu.async_copy(
                        tail_hbm.at[pl.ds(i * 8, 8)],
                        tbuf.at[pl.ds(i * 8, 8)], sr[p])

        def read_wait(k, p):
            tc = wid + 32 * k

            @pl.when(tc < _TCOLS)
            def _():
                for c_abs in range(_D):
                    pltpu.make_async_copy(
                        wt_hbm.at[c_abs, pl.ds(tc * 128, 128)],
                        buf[p].at[c_abs, pl.ds(0, 128)], sr[p]).wait()

            @pl.when(tc == _TCOLS)
            def _():
                for i in range(2):
                    pltpu.make_async_copy(
                        tail_hbm.at[pl.ds(i * 8, 8)],
                        tbuf.at[pl.ds(i * 8, 8)], sr[p]).wait()

        def transpose(p):
            bp, t1, tp = buf[p], tmid[p], tout[p]

            def cbody(c, _):
                cv = zeros16 + c
                for r8 in range(8):
                    v = bp[c, pl.ds(r8 * 16, 16)]
                    plsc.store_scatter(t1, [rlv[r8], cv], v)
                return 0
            lax.fori_loop(0, _D, cbody, 0, unroll=2)

            def mbody(m, _):
                m4 = m * 4
                for j in range(4):
                    for h in range(2):
                        v = t1[m4 + j, pl.ds(h * 16, 16)]
                        tp[m, pl.ds(j * 32 + h * 16, 16)] = v
                return 0
            lax.fori_loop(0, 32, mbody, 0, unroll=2)

        def write_start(k, p):
            tc = wid + 32 * k

            @pl.when(tc < _TCOLS)
            def _():
                pltpu.async_copy(
                    tout[p], lin_hbm.at[pl.ds(tc * 32, 32)], sw[p])

            @pl.when(tc == _TCOLS)
            def _():
                pltpu.async_copy(
                    tbuf, lin_hbm.at[pl.ds(tc * 32, _TAIL // 4)], sw[p])

        def write_wait(k, p):
            tc = wid + 32 * k

            @pl.when(tc < _TCOLS)
            def _():
                pltpu.make_async_copy(
                    tout[p], lin_hbm.at[pl.ds(tc * 32, 32)], sw[p]).wait()

            @pl.when(tc == _TCOLS)
            def _():
                pltpu.make_async_copy(
                    tbuf, lin_hbm.at[pl.ds(tc * 32, _TAIL // 4)], sw[p]).wait()

        nk = 245

        def in_range(k):
            return (k < nk) & (wid + 32 * k < ntc)

        read_start(0, 0)

        def step(j, _):
            for p in (0, 1):
                k = 2 * j + p

                @pl.when(in_range(k + 1))
                def _():
                    read_start(k + 1, 1 - p)

                @pl.when(in_range(k))
                def _():
                    read_wait(k, p)

                @pl.when((k >= 2) & in_range(k - 2))
                def _():
                    write_wait(k - 2, p)

                @pl.when(in_range(k) & (wid + 32 * k < _TCOLS))
                def _():
                    transpose(p)

                @pl.when(in_range(k))
                def _():
                    write_start(k, p)
            return 0
        lax.fori_loop(0, nit // 2, step, 0)

        @pl.when(in_range(nk - 1))
        def _():
            write_wait(nk - 1, (nk - 1) % 2)

    return retile_kernel


def _make_gather():
    mesh = plsc.VectorSubcoreMesh(core_axis_name="c", subcore_axis_name="s")

    @functools.partial(
        pl.kernel,
        mesh=mesh,
        out_type=jax.ShapeDtypeStruct((_SEQ, _D // 8, _NW, 8, _BLK),
                                      jnp.float32),
        compiler_params=pltpu.CompilerParams(use_tc_tiling_on_sc=False,
                                             needs_layout_passes=False),
        scratch_types=[
            pltpu.VMEM((_ROWS,), jnp.int32),
            pltpu.VMEM((_ROWS,), jnp.int32),
            pltpu.VMEM((_ROWS, _D), jnp.float32),
            pltpu.VMEM((_ROWS, _D), jnp.float32),
            pltpu.VMEM((_S, _D, _BLK + 1), jnp.float32),
            pltpu.VMEM((_S, _D, _BLK + 1), jnp.float32),
            pltpu.SemaphoreType.DMA,
            pltpu.SemaphoreType.DMA,
            pltpu.SemaphoreType.DMA,
            pltpu.SemaphoreType.DMA,
            pltpu.SemaphoreType.DMA,
            pltpu.SemaphoreType.DMA,
        ],
    )
    def gather_kernel(tt_hbm, table_hbm, out_hbm,
                      idx0, idx1, g0, g1, o0, o1,
                      si0, si1, sg0, sg1, sw0, sw1):
        wid = lax.axis_index("s") * 2 + lax.axis_index("c")
        col0 = wid * _BLK
        idx = (idx0, idx1)
        g = (g0, g1)
        o = (o0, o1)
        si = (si0, si1)
        sg = (sg0, sg1)
        sw = (sw0, sw1)

        def idx_start(i, p):
            for s_l in range(_S):
                pltpu.async_copy(
                    tt_hbm.at[i * _S + s_l, pl.ds(col0, _BLK)],
                    idx[p].at[pl.ds(s_l * _BLK, _BLK)], si[p])

        def idx_wait(i, p):
            for s_l in range(_S):
                pltpu.make_async_copy(
                    tt_hbm.at[i * _S + s_l, pl.ds(col0, _BLK)],
                    idx[p].at[pl.ds(s_l * _BLK, _BLK)], si[p]).wait()

        def gather_start(p):
            pltpu.async_copy(table_hbm.at[idx[p]], g[p], sg[p])

        def gather_wait(p):
            pltpu.make_async_copy(table_hbm.at[idx[p]], g[p], sg[p]).wait()

        lane = jax.lax.iota(jnp.int32, 16)
        zeros16 = jnp.zeros((16,), jnp.int32)
        c_lo = lane
        c_hi = lane + 16

        def transpose(p):
            gp, op = g[p], o[p]
            for s_l in range(_S):
                s_vec = zeros16 + s_l

                def bbody(bl, _):
                    row = s_l * _BLK + bl
                    v0 = gp[row, pl.ds(0, 16)]
                    v1 = gp[row, pl.ds(16, 16)]
                    b_vec = zeros16 + bl
                    plsc.store_scatter(op, [s_vec, c_lo, b_vec], v0)
                    plsc.store_scatter(op, [s_vec, c_hi, b_vec], v1)
                    return 0
                lax.fori_loop(0, _BLK, bbody, 0, unroll=8)

        def write_start(i, p):
            for s_l in range(_S):
                for tr in range(_D // 8):
                    pltpu.async_copy(
                        o[p].at[s_l, pl.ds(tr * 8, 8), pl.ds(0, _BLK)],
                        out_hbm.at[i * _S + s_l, tr, wid], sw[p])

        def write_wait(i, p):
            for s_l in range(_S):
                for tr in range(_D // 8):
                    pltpu.make_async_copy(
                        o[p].at[s_l, pl.ds(tr * 8, 8), pl.ds(0, _BLK)],
                        out_hbm.at[i * _S + s_l, tr, wid], sw[p]).wait()

        idx_start(0, 0)
        idx_start(1, 1)
        idx_wait(0, 0)
        gather_start(0)

        def step(k, _):
            for p in (0, 1):
                i = 2 * k + p
                gather_wait(p)

                @pl.when(i + 2 < _NIT)
                def _():
                    idx_start(i + 2, p)

                @pl.when(i + 1 < _NIT)
                def _():
                    idx_wait(i + 1, 1 - p)
                    gather_start(1 - p)

                @pl.when(i >= 2)
                def _():
                    write_wait(i - 2, p)

                transpose(p)
                write_start(i, p)
            return 0
        lax.fori_loop(0, _NIT // 2, step, 0)

        write_wait(_NIT - 2, 0)
        write_wait(_NIT - 1, 1)

    return gather_kernel


_retile = _make_retile()
_gather = _make_gather()


def kernel(token_ids, weight):
    wt = weight.T
    tail = weight[_TCOLS * 128:, :].reshape(_TAIL // 4, 128)
    lin = _retile(wt, tail)
    table = lin.reshape(_V, _D)
    tt = token_ids.T.astype(jnp.int32)
    x = _gather(tt, table)
    return x.transpose(2, 4, 0, 1, 3).reshape(_BATCH, _SEQ, _D)


# speedup vs baseline: 1.5955x; 1.5955x over previous
import functools

import jax
import jax.numpy as jnp
from jax import lax
from jax.experimental import pallas as pl
from jax.experimental.pallas import tpu as pltpu
from jax.experimental.pallas import tpu_sc as plsc

_D = 32
_SEQ = 200
_BATCH = 4096
_NW = 32
_BLK = _BATCH // _NW
_S = 5
_NIT = _SEQ // _S
_ROWS = _S * _BLK


def _make_gather():
    mesh = plsc.VectorSubcoreMesh(core_axis_name="c", subcore_axis_name="s")

    @functools.partial(
        pl.kernel,
        mesh=mesh,
        out_type=jax.ShapeDtypeStruct((_SEQ, _D // 8, _NW, 8, _BLK),
                                      jnp.float32),
        compiler_params=pltpu.CompilerParams(use_tc_tiling_on_sc=False,
                                             needs_layout_passes=False),
        scratch_types=[
            pltpu.VMEM((_ROWS,), jnp.int32),
            pltpu.VMEM((_ROWS,), jnp.int32),
            pltpu.VMEM((_ROWS, _D), jnp.float32),
            pltpu.VMEM((_ROWS, _D), jnp.float32),
            pltpu.VMEM((_S, _D, _BLK + 1), jnp.float32),
            pltpu.VMEM((_S, _D, _BLK + 1), jnp.float32),
            pltpu.SemaphoreType.DMA,
            pltpu.SemaphoreType.DMA,
            pltpu.SemaphoreType.DMA,
            pltpu.SemaphoreType.DMA,
            pltpu.SemaphoreType.DMA,
            pltpu.SemaphoreType.DMA,
        ],
    )
    def gather_kernel(tt_hbm, table_hbm, out_hbm,
                      idx0, idx1, g0, g1, o0, o1,
                      si0, si1, sg0, sg1, sw0, sw1):
        wid = lax.axis_index("s") * 2 + lax.axis_index("c")
        col0 = wid * _BLK
        idx = (idx0, idx1)
        g = (g0, g1)
        o = (o0, o1)
        si = (si0, si1)
        sg = (sg0, sg1)
        sw = (sw0, sw1)

        def idx_start(i, p):
            for s_l in range(_S):
                pltpu.async_copy(
                    tt_hbm.at[i * _S + s_l, pl.ds(col0, _BLK)],
                    idx[p].at[pl.ds(s_l * _BLK, _BLK)], si[p])

        def idx_wait(i, p):
            for s_l in range(_S):
                pltpu.make_async_copy(
                    tt_hbm.at[i * _S + s_l, pl.ds(col0, _BLK)],
                    idx[p].at[pl.ds(s_l * _BLK, _BLK)], si[p]).wait()

        def gather_start(p):
            pltpu.async_copy(table_hbm.at[idx[p]], g[p], sg[p])

        def gather_wait(p):
            pltpu.make_async_copy(table_hbm.at[idx[p]], g[p], sg[p]).wait()

        lane = jax.lax.iota(jnp.int32, 16)
        zeros16 = jnp.zeros((16,), jnp.int32)
        c_lo = lane
        c_hi = lane + 16

        def transpose(p):
            gp, op = g[p], o[p]
            for s_l in range(_S):
                s_vec = zeros16 + s_l

                def bbody(bl, _):
                    row = s_l * _BLK + bl
                    v0 = gp[row, pl.ds(0, 16)]
                    v1 = gp[row, pl.ds(16, 16)]
                    b_vec = zeros16 + bl
                    plsc.store_scatter(op, [s_vec, c_lo, b_vec], v0)
                    plsc.store_scatter(op, [s_vec, c_hi, b_vec], v1)
                    return 0
                lax.fori_loop(0, _BLK, bbody, 0, unroll=8)

        def write_start(i, p):
            for s_l in range(_S):
                for tr in range(_D // 8):
                    pltpu.async_copy(
                        o[p].at[s_l, pl.ds(tr * 8, 8), pl.ds(0, _BLK)],
                        out_hbm.at[i * _S + s_l, tr, wid], sw[p])

        def write_wait(i, p):
            for s_l in range(_S):
                for tr in range(_D // 8):
                    pltpu.make_async_copy(
                        o[p].at[s_l, pl.ds(tr * 8, 8), pl.ds(0, _BLK)],
                        out_hbm.at[i * _S + s_l, tr, wid], sw[p]).wait()

        idx_start(0, 0)
        idx_start(1, 1)
        idx_wait(0, 0)
        gather_start(0)

        def step(k, _):
            for p in (0, 1):
                i = 2 * k + p
                gather_wait(p)

                @pl.when(i + 2 < _NIT)
                def _():
                    idx_start(i + 2, p)

                @pl.when(i + 1 < _NIT)
                def _():
                    idx_wait(i + 1, 1 - p)
                    gather_start(1 - p)

                @pl.when(i >= 2)
                def _():
                    write_wait(i - 2, p)

                transpose(p)
                write_start(i, p)
            return 0
        lax.fori_loop(0, _NIT // 2, step, 0)

        write_wait(_NIT - 2, 0)
        write_wait(_NIT - 1, 1)

    return gather_kernel


_gather = _make_gather()


def kernel(token_ids, weight):
    tt = token_ids.T.astype(jnp.int32)
    x = _gather(tt, weight)
    return x.transpose(2, 4, 0, 1, 3).reshape(_BATCH, _SEQ, _D)
